# baseline (speedup 1.0000x reference)
import jax
import jax.numpy as jnp
from jax.experimental import pallas as pl
from jax.experimental.pallas import tpu as pltpu

_G = 16


def _enhance_kernel(idx_ref, par_ref, x_ref, out_ref):
    g_count, R, Dd = x_ref.shape
    b0 = pl.program_id(0) * g_count

    thr = jax.nn.sigmoid(par_ref[0])
    strength = jax.nn.sigmoid(par_ref[1]) * 0.2
    scale = par_ref[2]
    temp = jnp.clip(par_ref[3], 0.1, 10.0)
    inv_temp = 1.0 / temp

    out_ref[...] = x_ref[...]
    ones_col = jnp.ones((Dd, 1), dtype=jnp.float32)

    for g in range(g_count):
        qi = idx_ref[b0 + g]
        xg = x_ref[g]
        q = x_ref[g, pl.ds(qi, 1), :]
        qnorm = jnp.sqrt(jnp.sum(q * q))
        q_col = jnp.transpose(q)

        xx = xg * xg
        dots = jax.lax.dot_general(
            xg, q_col, (((1,), (0,)), ((), ())),
            preferred_element_type=jnp.float32)
        norms2 = jax.lax.dot_general(
            xx, ones_col, (((1,), (0,)), ((), ())),
            preferred_element_type=jnp.float32)

        dots_t = jnp.transpose(dots)
        nrm_t = jnp.sqrt(jnp.transpose(norms2))
        denom = jnp.maximum(nrm_t, 1e-12) * jnp.maximum(qnorm, 1e-12)
        sims = dots_t / denom

        col_ids = jax.lax.broadcasted_iota(jnp.int32, (1, R), 1)
        not_self = col_ids != qi
        valid = jnp.logical_and(sims > thr, not_self)
        sw = jax.nn.sigmoid((sims - thr) * 10.0)
        e = jnp.where(
            valid,
            jnp.exp((sims - 1.0) * inv_temp) * sw * (1.0 + scale * sims),
            0.0)
        s_sum = jnp.sum(e)
        has_valid = jnp.any(valid)

        v = jax.lax.dot_general(
            e, xg, (((1,), (0,)), ((), ())),
            preferred_element_type=jnp.float32)
        enhanced = (1.0 - strength) * q + strength * (v / (s_sum + 1e-8))
        new_q = jnp.where(has_valid, enhanced, q)
        out_ref[g, pl.ds(qi, 1), :] = new_q


def kernel(final_relation_representations, query_rels, similarity_threshold_raw,
           enhancement_strength_raw, similarity_weight_scale, temperature):
    x = final_relation_representations
    B, R, D = x.shape
    idx = query_rels.astype(jnp.int32)
    params = jnp.stack([
        similarity_threshold_raw.astype(jnp.float32),
        enhancement_strength_raw.astype(jnp.float32),
        similarity_weight_scale.astype(jnp.float32),
        temperature.astype(jnp.float32),
    ])

    grid = (B // _G,)
    out = pl.pallas_call(
        _enhance_kernel,
        grid_spec=pltpu.PrefetchScalarGridSpec(
            num_scalar_prefetch=2,
            grid=grid,
            in_specs=[
                pl.BlockSpec((_G, R, D), lambda i, idx_ref, par_ref: (i, 0, 0)),
            ],
            out_specs=pl.BlockSpec((_G, R, D), lambda i, idx_ref, par_ref: (i, 0, 0)),
        ),
        out_shape=jax.ShapeDtypeStruct((B, R, D), jnp.float32),
        compiler_params=pltpu.CompilerParams(
            dimension_semantics=("parallel",),
        ),
    )(idx, params, x)
    return out


# speedup vs baseline: 1.2467x; 1.2467x over previous
import jax
import jax.numpy as jnp
from jax.experimental import pallas as pl
from jax.experimental.pallas import tpu as pltpu

_G = 8


def _enhance_kernel(idx_ref, par_ref, x_ref, out_ref):
    g_count, R, Dd = x_ref.shape
    b0 = pl.program_id(0) * g_count

    thr = jax.nn.sigmoid(par_ref[0])
    strength = jax.nn.sigmoid(par_ref[1]) * 0.2
    scale = par_ref[2]
    temp = jnp.clip(par_ref[3], 0.1, 10.0)
    inv_temp = 1.0 / temp

    out_ref[...] = x_ref[...]
    ones_row = jnp.ones((1, Dd), dtype=jnp.float32)

    for g in range(g_count):
        qi = idx_ref[b0 + g]
        xg = x_ref[g]
        q = x_ref[g, pl.ds(qi, 1), :]
        qnorm = jnp.sqrt(jnp.sum(q * q))

        xt = jnp.transpose(xg)
        xxt = xt * xt
        dots_t = jax.lax.dot_general(
            q, xt, (((1,), (0,)), ((), ())),
            preferred_element_type=jnp.float32)
        norms2_t = jax.lax.dot_general(
            ones_row, xxt, (((1,), (0,)), ((), ())),
            preferred_element_type=jnp.float32)

        denom = jnp.maximum(jnp.sqrt(norms2_t), 1e-12) * jnp.maximum(qnorm, 1e-12)
        sims = dots_t / denom

        col_ids = jax.lax.broadcasted_iota(jnp.int32, (1, R), 1)
        not_self = col_ids != qi
        valid = jnp.logical_and(sims > thr, not_self)
        sw = jax.nn.sigmoid((sims - thr) * 10.0)
        e = jnp.where(
            valid,
            jnp.exp((sims - 1.0) * inv_temp) * sw * (1.0 + scale * sims),
            0.0)
        s_sum = jnp.sum(e)
        has_valid = jnp.any(valid)

        v = jax.lax.dot_general(
            e, xg, (((1,), (0,)), ((), ())),
            preferred_element_type=jnp.float32)
        enhanced = (1.0 - strength) * q + strength * (v / (s_sum + 1e-8))
        new_q = jnp.where(has_valid, enhanced, q)
        out_ref[g, pl.ds(qi, 1), :] = new_q


def kernel(final_relation_representations, query_rels, similarity_threshold_raw,
           enhancement_strength_raw, similarity_weight_scale, temperature):
    x = final_relation_representations
    B, R, D = x.shape
    idx = query_rels.astype(jnp.int32)
    params = jnp.stack([
        similarity_threshold_raw.astype(jnp.float32),
        enhancement_strength_raw.astype(jnp.float32),
        similarity_weight_scale.astype(jnp.float32),
        temperature.astype(jnp.float32),
    ])

    grid = (B // _G,)
    out = pl.pallas_call(
        _enhance_kernel,
        grid_spec=pltpu.PrefetchScalarGridSpec(
            num_scalar_prefetch=2,
            grid=grid,
            in_specs=[
                pl.BlockSpec((_G, R, D), lambda i, idx_ref, par_ref: (i, 0, 0)),
            ],
            out_specs=pl.BlockSpec((_G, R, D), lambda i, idx_ref, par_ref: (i, 0, 0)),
        ),
        out_shape=jax.ShapeDtypeStruct((B, R, D), jnp.float32),
        compiler_params=pltpu.CompilerParams(
            dimension_semantics=("parallel",),
        ),
    )(idx, params, x)
    return out


# speedup vs baseline: 1.2967x; 1.0401x over previous
import jax
import jax.numpy as jnp
from jax.experimental import pallas as pl
from jax.experimental.pallas import tpu as pltpu

_G = 16


def _enhance_kernel(idx_ref, par_ref, x_ref, out_ref):
    g_count, R, Dd = x_ref.shape
    b0 = pl.program_id(0) * g_count

    thr = jax.nn.sigmoid(par_ref[0])
    strength = jax.nn.sigmoid(par_ref[1]) * 0.2
    scale = par_ref[2]
    temp = jnp.clip(par_ref[3], 0.1, 10.0)
    inv_temp = 1.0 / temp

    out_ref[...] = x_ref[...]
    ones_row = jnp.ones((1, Dd), dtype=jnp.float32)

    for g in range(g_count):
        qi = idx_ref[b0 + g]
        xg = x_ref[g]
        q = x_ref[g, pl.ds(qi, 1), :]
        qnorm = jnp.sqrt(jnp.sum(q * q))

        xt = jnp.transpose(xg)
        xxt = xt * xt
        dots_t = jax.lax.dot_general(
            q, xt, (((1,), (0,)), ((), ())),
            preferred_element_type=jnp.float32)
        norms2_t = jax.lax.dot_general(
            ones_row, xxt, (((1,), (0,)), ((), ())),
            preferred_element_type=jnp.float32)

        denom = jnp.maximum(jnp.sqrt(norms2_t), 1e-12) * jnp.maximum(qnorm, 1e-12)
        sims = dots_t / denom

        col_ids = jax.lax.broadcasted_iota(jnp.int32, (1, R), 1)
        not_self = col_ids != qi
        valid = jnp.logical_and(sims > thr, not_self)
        sw = jax.nn.sigmoid((sims - thr) * 10.0)
        e = jnp.where(
            valid,
            jnp.exp((sims - 1.0) * inv_temp) * sw * (1.0 + scale * sims),
            0.0)
        s_sum = jnp.sum(e)
        has_valid = jnp.any(valid)

        v = jax.lax.dot_general(
            e, xg, (((1,), (0,)), ((), ())),
            preferred_element_type=jnp.float32)
        enhanced = (1.0 - strength) * q + strength * (v / (s_sum + 1e-8))
        new_q = jnp.where(has_valid, enhanced, q)
        out_ref[g, pl.ds(qi, 1), :] = new_q


def kernel(final_relation_representations, query_rels, similarity_threshold_raw,
           enhancement_strength_raw, similarity_weight_scale, temperature):
    x = final_relation_representations
    B, R, D = x.shape
    idx = query_rels.astype(jnp.int32)
    params = jnp.stack([
        similarity_threshold_raw.astype(jnp.float32),
        enhancement_strength_raw.astype(jnp.float32),
        similarity_weight_scale.astype(jnp.float32),
        temperature.astype(jnp.float32),
    ])

    grid = (B // _G,)
    out = pl.pallas_call(
        _enhance_kernel,
        grid_spec=pltpu.PrefetchScalarGridSpec(
            num_scalar_prefetch=2,
            grid=grid,
            in_specs=[
                pl.BlockSpec((_G, R, D), lambda i, idx_ref, par_ref: (i, 0, 0)),
            ],
            out_specs=pl.BlockSpec((_G, R, D), lambda i, idx_ref, par_ref: (i, 0, 0)),
        ),
        out_shape=jax.ShapeDtypeStruct((B, R, D), jnp.float32),
        compiler_params=pltpu.CompilerParams(
            dimension_semantics=("parallel",),
        ),
    )(idx, params, x)
    return out
